# baseline (speedup 1.0000x reference)
import jax
import jax.numpy as jnp
from jax import lax
from jax.experimental import pallas as pl
from jax.experimental.pallas import tpu as pltpu

_TM = 400


def _xw_body(x_ref, w1_ref, y_ref):
    y_ref[...] = jnp.dot(
        x_ref[...].astype(jnp.bfloat16),
        w1_ref[...],
        preferred_element_type=jnp.float32,
    ).astype(jnp.bfloat16)


def _fused_body(adj_ref, y_ref, b1_ref, w2_ref, b2_ref,
                out_ref, adjq_hbm,
                h2_ref, csum_ref, stage_ref, sems):
    p = pl.program_id(0)
    i = pl.program_id(1)
    ni = pl.num_programs(1)
    slot = lax.rem(i, 2)

    @pl.when(p == 0)
    def _layer1():
        a = adj_ref[...]

        @pl.when(i >= 2)
        def _wait_prev():
            pltpu.make_async_copy(
                stage_ref.at[slot], adjq_hbm.at[i - 2], sems.at[slot]
            ).wait()

        qi = (a * 254.0 + 0.5).astype(jnp.int32)
        stage_ref[slot] = (qi - 127).astype(jnp.int8)
        pltpu.make_async_copy(
            stage_ref.at[slot], adjq_hbm.at[i], sems.at[slot]
        ).start()

        acc = jnp.dot(
            a.astype(jnp.bfloat16),
            y_ref[...],
            preferred_element_type=jnp.float32,
        )
        h = jnp.maximum(acc + b1_ref[...], 0.0).astype(jnp.bfloat16)
        h2_ref[pl.ds(i * _TM, _TM), :] = jnp.dot(
            h, w2_ref[...], preferred_element_type=jnp.float32
        ).astype(jnp.bfloat16)

    @pl.when((p == 1) & (i == 0))
    def _transition():
        pltpu.make_async_copy(
            stage_ref.at[(ni - 2) % 2], adjq_hbm.at[ni - 2],
            sems.at[(ni - 2) % 2]
        ).wait()
        pltpu.make_async_copy(
            stage_ref.at[(ni - 1) % 2], adjq_hbm.at[ni - 1],
            sems.at[(ni - 1) % 2]
        ).wait()
        csum_ref[...] = jnp.sum(
            h2_ref[...].astype(jnp.float32), axis=0, keepdims=True
        )
        pltpu.make_async_copy(
            adjq_hbm.at[0], stage_ref.at[0], sems.at[0]).start()
        pltpu.make_async_copy(
            adjq_hbm.at[1], stage_ref.at[1], sems.at[1]).start()

    @pl.when(p == 1)
    def _layer2():
        pltpu.make_async_copy(
            adjq_hbm.at[i], stage_ref.at[slot], sems.at[slot]
        ).wait()
        qa = stage_ref[slot].astype(jnp.bfloat16)
        acc = jnp.dot(qa, h2_ref[...], preferred_element_type=jnp.float32)
        out_ref[...] = acc * (1.0 / 254.0) + \
            (127.0 / 254.0) * csum_ref[...] + b2_ref[...]

        @pl.when(i + 2 < ni)
        def _prefetch_next():
            pltpu.make_async_copy(
                adjq_hbm.at[i + 2], stage_ref.at[slot], sems.at[slot]
            ).start()


def kernel(x, adj, W1, b1, W2, b2):
    n, fin = x.shape
    h_dim = W1.shape[1]
    fout = W2.shape[1]
    ni = n // _TM

    y = pl.pallas_call(
        _xw_body,
        grid=(n // 2000,),
        in_specs=[
            pl.BlockSpec((2000, fin), lambda i: (i, 0)),
            pl.BlockSpec((fin, h_dim), lambda i: (0, 0)),
        ],
        out_specs=pl.BlockSpec((2000, h_dim), lambda i: (i, 0)),
        out_shape=jax.ShapeDtypeStruct((n, h_dim), jnp.bfloat16),
    )(x, W1.astype(jnp.bfloat16))

    out, _ = pl.pallas_call(
        _fused_body,
        grid=(2, ni),
        in_specs=[
            pl.BlockSpec(
                (_TM, n), lambda p, i: (jnp.where(p == 0, i, ni - 1), 0)),
            pl.BlockSpec((n, h_dim), lambda p, i: (0, 0)),
            pl.BlockSpec((1, h_dim), lambda p, i: (0, 0)),
            pl.BlockSpec((h_dim, h_dim), lambda p, i: (0, 0)),
            pl.BlockSpec((1, fout), lambda p, i: (0, 0)),
        ],
        out_specs=[
            pl.BlockSpec(
                (_TM, fout), lambda p, i: (jnp.where(p == 0, 0, i), 0)),
            pl.BlockSpec(memory_space=pl.ANY),
        ],
        out_shape=[
            jax.ShapeDtypeStruct((n, fout), jnp.float32),
            jax.ShapeDtypeStruct((ni, _TM, n), jnp.int8),
        ],
        scratch_shapes=[
            pltpu.VMEM((n, h_dim), jnp.bfloat16),
            pltpu.VMEM((1, h_dim), jnp.float32),
            pltpu.VMEM((2, _TM, n), jnp.int8),
            pltpu.SemaphoreType.DMA((2,)),
        ],
        compiler_params=pltpu.CompilerParams(
            dimension_semantics=("arbitrary", "arbitrary"),
            vmem_limit_bytes=64 * 1024 * 1024,
        ),
    )(adj, y, b1.reshape(1, h_dim), W2.astype(jnp.bfloat16),
      b2.reshape(1, fout))

    return out


# speedup vs baseline: 1.3208x; 1.3208x over previous
import jax
import jax.numpy as jnp
from jax.experimental import pallas as pl
from jax.experimental.pallas import tpu as pltpu

_TM = 400


def _pass_a_body(adj_ref, x_ref, w1_ref, b1_ref, w2_ref,
                 h2_ref, adjq_ref, csum_ref, y_ref, cacc_ref):
    @pl.when(pl.program_id(0) == 0)
    def _compute_y():
        y_ref[...] = jnp.dot(
            x_ref[...].astype(jnp.bfloat16),
            w1_ref[...],
            preferred_element_type=jnp.float32,
        ).astype(jnp.bfloat16)
        cacc_ref[...] = jnp.zeros_like(cacc_ref)

    a = adj_ref[...]
    qi = (a * 254.0 + 0.5).astype(jnp.int32)
    adjq_ref[...] = (qi - 127).astype(jnp.int8)[None]

    acc = jnp.dot(
        a.astype(jnp.bfloat16),
        y_ref[...],
        preferred_element_type=jnp.float32,
    )
    h = jnp.maximum(acc + b1_ref[...], 0.0).astype(jnp.bfloat16)
    h2b = jnp.dot(
        h, w2_ref[...], preferred_element_type=jnp.float32
    ).astype(jnp.bfloat16)
    h2_ref[...] = h2b
    cacc_ref[...] += jnp.sum(
        h2b.astype(jnp.float32), axis=0, keepdims=True)
    csum_ref[...] = cacc_ref[...]


def _pass_b_body(adjq_ref, h2_ref, csum_ref, b2_ref, out_ref):
    qa = adjq_ref[0].astype(jnp.bfloat16)
    acc = jnp.dot(qa, h2_ref[...], preferred_element_type=jnp.float32)
    out_ref[...] = acc * (1.0 / 254.0) + \
        (127.0 / 254.0) * csum_ref[...] + b2_ref[...]


def kernel(x, adj, W1, b1, W2, b2):
    n, fin = x.shape
    h_dim = W1.shape[1]
    fout = W2.shape[1]
    ni = n // _TM

    h2, adjq, csum = pl.pallas_call(
        _pass_a_body,
        grid=(ni,),
        in_specs=[
            pl.BlockSpec((_TM, n), lambda i: (i, 0)),
            pl.BlockSpec((n, fin), lambda i: (0, 0)),
            pl.BlockSpec((fin, h_dim), lambda i: (0, 0)),
            pl.BlockSpec((1, h_dim), lambda i: (0, 0)),
            pl.BlockSpec((h_dim, h_dim), lambda i: (0, 0)),
        ],
        out_specs=[
            pl.BlockSpec((_TM, h_dim), lambda i: (i, 0)),
            pl.BlockSpec((1, _TM, n), lambda i: (i, 0, 0)),
            pl.BlockSpec((1, h_dim), lambda i: (0, 0)),
        ],
        out_shape=[
            jax.ShapeDtypeStruct((n, h_dim), jnp.bfloat16),
            jax.ShapeDtypeStruct((ni, _TM, n), jnp.int8),
            jax.ShapeDtypeStruct((1, h_dim), jnp.float32),
        ],
        scratch_shapes=[
            pltpu.VMEM((n, h_dim), jnp.bfloat16),
            pltpu.VMEM((1, h_dim), jnp.float32),
        ],
        compiler_params=pltpu.CompilerParams(
            dimension_semantics=("arbitrary",),
            vmem_limit_bytes=64 * 1024 * 1024,
        ),
    )(adj, x, W1.astype(jnp.bfloat16), b1.reshape(1, h_dim),
      W2.astype(jnp.bfloat16))

    out = pl.pallas_call(
        _pass_b_body,
        grid=(ni,),
        in_specs=[
            pl.BlockSpec((1, _TM, n), lambda i: (i, 0, 0)),
            pl.BlockSpec((n, h_dim), lambda i: (0, 0)),
            pl.BlockSpec((1, h_dim), lambda i: (0, 0)),
            pl.BlockSpec((1, fout), lambda i: (0, 0)),
        ],
        out_specs=pl.BlockSpec((_TM, fout), lambda i: (i, 0)),
        out_shape=jax.ShapeDtypeStruct((n, fout), jnp.float32),
        compiler_params=pltpu.CompilerParams(
            dimension_semantics=("arbitrary",),
            vmem_limit_bytes=64 * 1024 * 1024,
        ),
    )(adjq, h2, csum, b2.reshape(1, fout))

    return out
